# baseline (speedup 1.0000x reference)
import functools

import jax
import jax.numpy as jnp
from jax import lax
from jax.experimental import pallas as pl
from jax.experimental.pallas import tpu as pltpu
from jax.experimental.pallas import tpu_sc as plsc

NUM_STOCKS = 100000
CELL_SIZE = 64
BATCH = 16384
ROWS = NUM_STOCKS + 1

NC, NS = 2, 16
NW = NC * NS
B_PER_W = BATCH // NW
CHUNK = 128
NCHUNK = B_PER_W // CHUNK

PACK_BC = 512


def _pack_tables(emb0T, emb1T):
    def body(a_ref, b_ref, o_ref):
        o_ref[:, :CELL_SIZE] = a_ref[...].T
        o_ref[:, CELL_SIZE:] = b_ref[...].T

    return pl.pallas_call(
        body,
        grid=(pl.cdiv(ROWS, PACK_BC),),
        in_specs=[
            pl.BlockSpec((CELL_SIZE, PACK_BC), lambda i: (0, i)),
            pl.BlockSpec((CELL_SIZE, PACK_BC), lambda i: (0, i)),
        ],
        out_specs=pl.BlockSpec((PACK_BC, 2 * CELL_SIZE), lambda i: (i, 0)),
        out_shape=jax.ShapeDtypeStruct((ROWS, 2 * CELL_SIZE), jnp.float32),
    )(emb0T, emb1T)


def _encoder_gather(idx_flat, packed):
    mesh = plsc.VectorSubcoreMesh(core_axis_name="c", subcore_axis_name="s")
    out_t = jax.ShapeDtypeStruct((BATCH, 2 * CELL_SIZE), jnp.float32)

    @functools.partial(
        pl.kernel,
        out_type=out_t,
        mesh=mesh,
        scratch_types=[
            pltpu.VMEM((B_PER_W,), jnp.int32),
            pltpu.VMEM((B_PER_W, 2 * CELL_SIZE), jnp.float32),
            pltpu.SemaphoreType.DMA,
            pltpu.SemaphoreType.DMA,
        ],
    )
    def k(tab_hbm, idx_hbm, o_hbm, idx_v, rows_v, sem_g, sem_w):
        wid = lax.axis_index("s") * NC + lax.axis_index("c")
        base = wid * B_PER_W
        pltpu.sync_copy(idx_hbm.at[pl.ds(base, B_PER_W)], idx_v)

        gathers = []
        for j in range(NCHUNK):
            sl = pl.ds(j * CHUNK, CHUNK)
            gathers.append(pltpu.async_copy(
                tab_hbm.at[idx_v.at[sl]], rows_v.at[sl], sem_g))
        for c in gathers:
            c.wait()
        pltpu.async_copy(rows_v, o_hbm.at[pl.ds(base, B_PER_W)], sem_w).wait()

    return k(packed, idx_flat)


def kernel(Stock_ID, emb0, emb1):
    idx_flat = Stock_ID.reshape(BATCH).astype(jnp.int32)
    packed = _pack_tables(emb0.T, emb1.T)
    out = _encoder_gather(idx_flat, packed)
    return (out[:, :CELL_SIZE], out[:, CELL_SIZE:])


# speedup vs baseline: 1.7204x; 1.7204x over previous
import functools

import jax
import jax.numpy as jnp
from jax import lax
from jax.experimental import pallas as pl
from jax.experimental.pallas import tpu as pltpu
from jax.experimental.pallas import tpu_sc as plsc

NUM_STOCKS = 100000
CELL_SIZE = 64
BATCH = 16384

NC, NS = 2, 16
NW = NC * NS
B_PER_W = BATCH // NW


def _gather_one(idx_flat, emb):
    mesh = plsc.VectorSubcoreMesh(core_axis_name="c", subcore_axis_name="s")
    out_t = jax.ShapeDtypeStruct((BATCH, CELL_SIZE), jnp.float32)

    @functools.partial(
        pl.kernel,
        out_type=out_t,
        mesh=mesh,
        scratch_types=[
            pltpu.VMEM((B_PER_W,), jnp.int32),
            pltpu.VMEM((B_PER_W, CELL_SIZE), jnp.float32),
            pltpu.SemaphoreType.DMA,
            pltpu.SemaphoreType.DMA,
        ],
    )
    def k(e_hbm, idx_hbm, o_hbm, idx_v, rows_v, sem_g, sem_w):
        wid = lax.axis_index("s") * NC + lax.axis_index("c")
        base = wid * B_PER_W
        pltpu.sync_copy(idx_hbm.at[pl.ds(base, B_PER_W)], idx_v)

        @pl.loop(0, B_PER_W, step=16)
        def _(j):
            v = idx_v[pl.ds(j, 16)]
            for t in range(16):
                pltpu.make_async_copy(
                    e_hbm.at[v[t]], rows_v.at[j + t], sem_g).start()

        pltpu.make_async_copy(
            o_hbm.at[pl.ds(base, B_PER_W)], rows_v, sem_g).wait()
        pltpu.async_copy(
            rows_v, o_hbm.at[pl.ds(base, B_PER_W)], sem_w).wait()

    return k(emb, idx_flat)


def kernel(Stock_ID, emb0, emb1):
    idx_flat = Stock_ID.reshape(BATCH).astype(jnp.int32)
    o0 = _gather_one(idx_flat, emb0)
    o1 = _gather_one(idx_flat, emb1)
    return (o0, o1)
